# baseline (speedup 1.0000x reference)
import functools

import jax
import jax.numpy as jnp
from jax import lax
from jax.experimental import pallas as pl
from jax.experimental.pallas import tpu as pltpu
from jax.experimental.pallas import tpu_sc as plsc

_D = 64
_NC = 2
_NS = 16
_NW = _NC * _NS
_KSUB = 128
_NSTREAM = 8
_CHUNK = _KSUB * _NSTREAM


@functools.cache
def _gather_call(n_rows: int):
    b_per_w = n_rows // _NW
    n_chunks = b_per_w // _CHUNK
    grp_per_w = b_per_w // _KSUB
    mesh = plsc.VectorSubcoreMesh(core_axis_name="c", subcore_axis_name="s")

    @functools.partial(
        pl.kernel,
        out_type=jax.ShapeDtypeStruct((n_rows, _D), jnp.float32),
        mesh=mesh,
        scratch_types=[
            pltpu.VMEM((_NSTREAM, _KSUB), jnp.int32),
            pltpu.VMEM((_CHUNK, _D), jnp.float32),
            pltpu.SemaphoreType.DMA,
        ],
        compiler_params=pltpu.CompilerParams(use_tc_tiling_on_sc=False),
    )
    def body(idx_hbm, table_hbm, out_hbm, idx_v, rows_v, gsem):
        wid = lax.axis_index("s") * _NC + lax.axis_index("c")
        row0 = wid * grp_per_w

        def chunk(j, carry):
            grp = pl.multiple_of(row0 + j * _NSTREAM, _NSTREAM)
            pltpu.sync_copy(idx_hbm.at[pl.ds(grp, _NSTREAM)], idx_v)
            copies = [
                pltpu.async_copy(
                    table_hbm.at[idx_v.at[t]],
                    rows_v.at[pl.ds(t * _KSUB, _KSUB)],
                    gsem,
                )
                for t in range(_NSTREAM)
            ]
            for cp in copies:
                cp.wait()
            base = pl.multiple_of(grp * _KSUB, _CHUNK)
            pltpu.sync_copy(rows_v, out_hbm.at[pl.ds(base, _CHUNK)])
            return carry

        lax.fori_loop(0, n_chunks, chunk, 0)

    return body


def kernel(input_ids, embed_table):
    ids = input_ids.reshape(-1).astype(jnp.int32)
    n_rows = ids.shape[0]
    idx2 = ids.reshape(n_rows // _KSUB, _KSUB)
    out = _gather_call(n_rows)(idx2, embed_table)
    return out.reshape(input_ids.shape + (_D,))


# speedup vs baseline: 1.0176x; 1.0176x over previous
import functools

import jax
import jax.numpy as jnp
from jax import lax
from jax.experimental import pallas as pl
from jax.experimental.pallas import tpu as pltpu
from jax.experimental.pallas import tpu_sc as plsc

_D = 64
_NC = 2
_NS = 16
_NW = _NC * _NS
_KSUB = 128
_NSTREAM = 4
_CHUNK = _KSUB * _NSTREAM


@functools.cache
def _gather_call(n_rows: int):
    b_per_w = n_rows // _NW
    grp_per_w = b_per_w // _KSUB
    n_chunks = b_per_w // _CHUNK
    n_super = n_chunks // 2
    mesh = plsc.VectorSubcoreMesh(core_axis_name="c", subcore_axis_name="s")

    @functools.partial(
        pl.kernel,
        out_type=jax.ShapeDtypeStruct((n_rows, _D), jnp.float32),
        mesh=mesh,
        scratch_types=[
            pltpu.VMEM((grp_per_w, _KSUB), jnp.int32),
            pltpu.VMEM((_CHUNK, _D), jnp.float32),
            pltpu.VMEM((_CHUNK, _D), jnp.float32),
            pltpu.SemaphoreType.DMA,
            pltpu.SemaphoreType.DMA,
            pltpu.SemaphoreType.DMA,
            pltpu.SemaphoreType.DMA,
        ],
        compiler_params=pltpu.CompilerParams(use_tc_tiling_on_sc=False),
    )
    def body(idx_hbm, table_hbm, out_hbm, idx_all, rows0, rows1,
             gsem0, gsem1, wsem0, wsem1):
        wid = lax.axis_index("s") * _NC + lax.axis_index("c")
        grp0 = pl.multiple_of(wid * grp_per_w, grp_per_w)
        out0 = pl.multiple_of(wid * b_per_w, b_per_w)
        rows = (rows0, rows1)
        gsems = (gsem0, gsem1)
        wsems = (wsem0, wsem1)

        pltpu.sync_copy(idx_hbm.at[pl.ds(grp0, grp_per_w)], idx_all)

        def fire_gather(j, b):
            for t in range(_NSTREAM):
                pltpu.async_copy(
                    table_hbm.at[idx_all.at[j * _NSTREAM + t]],
                    rows[b].at[pl.ds(t * _KSUB, _KSUB)],
                    gsems[b],
                )

        def wait_gather(b):
            pltpu.make_async_copy(
                table_hbm.at[pl.ds(0, _CHUNK)], rows[b], gsems[b]).wait()

        def fire_write(j, b):
            off = pl.multiple_of(out0 + j * _CHUNK, _CHUNK)
            pltpu.async_copy(rows[b], out_hbm.at[pl.ds(off, _CHUNK)], wsems[b])

        def wait_write(b):
            pltpu.make_async_copy(
                rows[b], out_hbm.at[pl.ds(0, _CHUNK)], wsems[b]).wait()

        fire_gather(0, 0)

        def super_body(s, carry):
            j = 2 * s

            @pl.when(s > 0)
            def _():
                wait_write(1)
            fire_gather(j + 1, 1)
            wait_gather(0)
            fire_write(j, 0)

            wait_write(0)

            @pl.when(s < n_super - 1)
            def _():
                fire_gather(j + 2, 0)
            wait_gather(1)
            fire_write(j + 1, 1)
            return carry

        lax.fori_loop(0, n_super, super_body, 0)
        wait_write(1)

    return body


def kernel(input_ids, embed_table):
    ids = input_ids.reshape(-1).astype(jnp.int32)
    n_rows = ids.shape[0]
    idx2 = ids.reshape(n_rows // _KSUB, _KSUB)
    out = _gather_call(n_rows)(idx2, embed_table)
    return out.reshape(input_ids.shape + (_D,))
